# baseline (speedup 1.0000x reference)
import functools

import jax
import jax.numpy as jnp
from jax import lax
from jax.experimental import pallas as pl
from jax.experimental.pallas import tpu as pltpu
from jax.experimental.pallas import tpu_sc as plsc

NUM_DRUGS = 1000000
UNKNOWN_PADDING = 100
TOTAL_VOCAB = NUM_DRUGS + UNKNOWN_PADDING
FEATURE_DIM = 64
FEATURE_PROJ_DIM = 256
IDENTITY_DIM = 256
FUSED_DIM = 512
BATCH = 16384

_NC = 2
_NS = 16
_NW = _NC * _NS
_NSLICE = 4
_SB = BATCH // _NSLICE
_CHUNK = _SB // _NW
_BM = 2048
_GRID = _SB // _BM


def _gather_body(idx_hbm, emb_hbm, out_hbm, idx_v, rows_v, sem):
  wid = lax.axis_index("s") * _NC + lax.axis_index("c")
  pltpu.sync_copy(idx_hbm.at[wid], idx_v)
  pltpu.async_copy(emb_hbm.at[idx_v.at[0]], rows_v, sem).wait()
  pltpu.sync_copy(rows_v, out_hbm.at[pl.ds(wid * _CHUNK, _CHUNK)])


@functools.lru_cache(maxsize=None)
def _build_gather():
  return pl.kernel(
      _gather_body,
      out_type=jax.ShapeDtypeStruct((_SB, IDENTITY_DIM), jnp.float32),
      mesh=plsc.VectorSubcoreMesh(
          core_axis_name="c", subcore_axis_name="s",
          num_cores=_NC, num_subcores=_NS),
      scratch_types=[
          pltpu.VMEM((1, _CHUNK), jnp.int32),
          pltpu.VMEM((_CHUNK, IDENTITY_DIM), jnp.float32),
          pltpu.SemaphoreType.DMA,
      ],
  )


def _mlp_body(ident_ref, feat_ref, wf_ref, bf_ref, w1_ref, b1_ref,
              gamma_ref, beta_ref, w2_ref, b2_ref, out_ref):
  fp = jnp.dot(feat_ref[...], wf_ref[...],
               preferred_element_type=jnp.float32) + bf_ref[...]
  h = (jnp.dot(ident_ref[...], w1_ref[:IDENTITY_DIM, :],
               preferred_element_type=jnp.float32)
       + jnp.dot(fp, w1_ref[IDENTITY_DIM:, :],
                 preferred_element_type=jnp.float32)
       + b1_ref[...])
  mean = jnp.mean(h, axis=-1, keepdims=True)
  var = jnp.mean(jnp.square(h - mean), axis=-1, keepdims=True)
  h = (h - mean) * lax.rsqrt(var + 1e-5)
  h = h * gamma_ref[...] + beta_ref[...]
  h = 0.5 * h * (1.0 + lax.erf(h * (2.0 ** -0.5)))
  out_ref[...] = jnp.dot(h, w2_ref[...],
                         preferred_element_type=jnp.float32) + b2_ref[...]


def _mlp_alias_body(prev_ref, *rest):
  del prev_ref
  _mlp_body(*rest)


def _full(shape):
  return pl.BlockSpec(shape, lambda i: (0,) * len(shape))


def _weight_specs():
  return [
      pl.BlockSpec((_BM, FEATURE_DIM), lambda i: (i, 0)),
      _full((FEATURE_DIM, FEATURE_PROJ_DIM)),
      _full((1, FEATURE_PROJ_DIM)),
      _full((IDENTITY_DIM + FEATURE_PROJ_DIM, FUSED_DIM)),
      _full((1, FUSED_DIM)),
      _full((1, FUSED_DIM)),
      _full((1, FUSED_DIM)),
      _full((FUSED_DIM, FUSED_DIM)),
      _full((1, FUSED_DIM)),
  ]


@functools.lru_cache(maxsize=None)
def _build_mlp(s):
  base = s * _GRID
  out_spec = pl.BlockSpec((_BM, FUSED_DIM), lambda i: (base + i, 0))
  ident_spec = pl.BlockSpec((_BM, IDENTITY_DIM), lambda i: (i, 0))
  if s == 0:
    return pl.pallas_call(
        _mlp_body,
        grid=(_GRID,),
        in_specs=[ident_spec] + _weight_specs(),
        out_specs=out_spec,
        out_shape=jax.ShapeDtypeStruct((BATCH, FUSED_DIM), jnp.float32),
        compiler_params=pltpu.CompilerParams(
            dimension_semantics=("arbitrary",)),
    )
  return pl.pallas_call(
      _mlp_alias_body,
      grid=(_GRID,),
      in_specs=([pl.BlockSpec(memory_space=pl.ANY), ident_spec]
                + _weight_specs()),
      out_specs=out_spec,
      out_shape=jax.ShapeDtypeStruct((BATCH, FUSED_DIM), jnp.float32),
      input_output_aliases={0: 0},
      compiler_params=pltpu.CompilerParams(
          dimension_semantics=("arbitrary",)),
  )


@jax.jit
def kernel(drug_id, features, emb, W_feat, b_feat, W1, b1, gamma, beta, W2, b2):
  safe_id = jnp.clip(drug_id, 0, TOTAL_VOCAB - 1)
  idx = safe_id.reshape(_NSLICE, _NW, 1, _CHUNK)
  gather = _build_gather()
  idents = [gather(idx[s], emb) for s in range(_NSLICE)]
  ws = (W_feat, b_feat.reshape(1, -1), W1, b1.reshape(1, -1),
        gamma.reshape(1, -1), beta.reshape(1, -1), W2, b2.reshape(1, -1))
  feats = features.reshape(_NSLICE, _SB, FEATURE_DIM)
  out = _build_mlp(0)(idents[0], feats[0], *ws)
  for s in range(1, _NSLICE):
    out = _build_mlp(s)(out, idents[s], feats[s], *ws)
  return out


# speedup vs baseline: 1.2998x; 1.2998x over previous
import functools

import jax
import jax.numpy as jnp
from jax import lax
from jax.experimental import pallas as pl
from jax.experimental.pallas import tpu as pltpu
from jax.experimental.pallas import tpu_sc as plsc

NUM_DRUGS = 1000000
UNKNOWN_PADDING = 100
TOTAL_VOCAB = NUM_DRUGS + UNKNOWN_PADDING
FEATURE_DIM = 64
FEATURE_PROJ_DIM = 256
IDENTITY_DIM = 256
FUSED_DIM = 512
BATCH = 16384

_NC = 2
_NS = 16
_NW = _NC * _NS
_BPW = BATCH // _NW
_CHUNK = 128
_NCHUNK = _BPW // _CHUNK


def _gather_body(idx_hbm, emb_hbm, out_hbm, idx_v, rows_a, rows_b, sem_a, sem_b):
  wid = lax.axis_index("s") * _NC + lax.axis_index("c")
  base = wid * _BPW
  pltpu.sync_copy(idx_hbm.at[wid], idx_v)
  rows = (rows_a, rows_b)
  sems = (sem_a, sem_b)
  copies = []
  for c in range(_NCHUNK):
    copies.append(
        pltpu.async_copy(emb_hbm.at[idx_v.at[c]], rows[c % 2], sems[c % 2]))
    if c >= 1:
      copies[c - 1].wait()
      pltpu.sync_copy(rows[(c - 1) % 2],
                      out_hbm.at[pl.ds(base + (c - 1) * _CHUNK, _CHUNK)])
  copies[_NCHUNK - 1].wait()
  pltpu.sync_copy(rows[(_NCHUNK - 1) % 2],
                  out_hbm.at[pl.ds(base + (_NCHUNK - 1) * _CHUNK, _CHUNK)])


@functools.lru_cache(maxsize=None)
def _build_gather():
  return pl.kernel(
      _gather_body,
      out_type=jax.ShapeDtypeStruct((BATCH, IDENTITY_DIM), jnp.float32),
      mesh=plsc.VectorSubcoreMesh(
          core_axis_name="c", subcore_axis_name="s",
          num_cores=_NC, num_subcores=_NS),
      scratch_types=[
          pltpu.VMEM((_NCHUNK, _CHUNK), jnp.int32),
          pltpu.VMEM((_CHUNK, IDENTITY_DIM), jnp.float32),
          pltpu.VMEM((_CHUNK, IDENTITY_DIM), jnp.float32),
          pltpu.SemaphoreType.DMA,
          pltpu.SemaphoreType.DMA,
      ],
  )

_BM = 2048


def _mlp_body(ident_ref, feat_ref, wf_ref, bf_ref, w1_ref, b1_ref,
              gamma_ref, beta_ref, w2_ref, b2_ref, out_ref):
  fp = jnp.dot(feat_ref[...], wf_ref[...],
               preferred_element_type=jnp.float32) + bf_ref[...]
  h = (jnp.dot(ident_ref[...], w1_ref[:IDENTITY_DIM, :],
               preferred_element_type=jnp.float32)
       + jnp.dot(fp, w1_ref[IDENTITY_DIM:, :],
                 preferred_element_type=jnp.float32)
       + b1_ref[...])
  mean = jnp.mean(h, axis=-1, keepdims=True)
  var = jnp.mean(jnp.square(h - mean), axis=-1, keepdims=True)
  h = (h - mean) * lax.rsqrt(var + 1e-5)
  h = h * gamma_ref[...] + beta_ref[...]
  h = 0.5 * h * (1.0 + lax.erf(h * (2.0 ** -0.5)))
  out_ref[...] = jnp.dot(h, w2_ref[...],
                         preferred_element_type=jnp.float32) + b2_ref[...]


def _full(shape):
  return pl.BlockSpec(shape, lambda i: (0,) * len(shape))


_mlp = pl.pallas_call(
    _mlp_body,
    grid=(BATCH // _BM,),
    in_specs=[
        pl.BlockSpec((_BM, IDENTITY_DIM), lambda i: (i, 0)),
        pl.BlockSpec((_BM, FEATURE_DIM), lambda i: (i, 0)),
        _full((FEATURE_DIM, FEATURE_PROJ_DIM)),
        _full((1, FEATURE_PROJ_DIM)),
        _full((IDENTITY_DIM + FEATURE_PROJ_DIM, FUSED_DIM)),
        _full((1, FUSED_DIM)),
        _full((1, FUSED_DIM)),
        _full((1, FUSED_DIM)),
        _full((FUSED_DIM, FUSED_DIM)),
        _full((1, FUSED_DIM)),
    ],
    out_specs=pl.BlockSpec((_BM, FUSED_DIM), lambda i: (i, 0)),
    out_shape=jax.ShapeDtypeStruct((BATCH, FUSED_DIM), jnp.float32),
    compiler_params=pltpu.CompilerParams(
        dimension_semantics=("parallel",)),
)


@jax.jit
def kernel(drug_id, features, emb, W_feat, b_feat, W1, b1, gamma, beta, W2, b2):
  safe_id = jnp.clip(drug_id, 0, TOTAL_VOCAB - 1)
  idx3 = safe_id.reshape(_NW, _NCHUNK, _CHUNK)
  identity = _build_gather()(idx3, emb)
  return _mlp(identity, features,
              W_feat, b_feat.reshape(1, -1),
              W1, b1.reshape(1, -1),
              gamma.reshape(1, -1), beta.reshape(1, -1),
              W2, b2.reshape(1, -1))


# speedup vs baseline: 1.3071x; 1.0056x over previous
import functools

import jax
import jax.numpy as jnp
from jax import lax
from jax.experimental import pallas as pl
from jax.experimental.pallas import tpu as pltpu
from jax.experimental.pallas import tpu_sc as plsc

NUM_DRUGS = 1000000
UNKNOWN_PADDING = 100
TOTAL_VOCAB = NUM_DRUGS + UNKNOWN_PADDING
FEATURE_DIM = 64
FEATURE_PROJ_DIM = 256
IDENTITY_DIM = 256
FUSED_DIM = 512
BATCH = 16384

_NC = 2
_NS = 16
_NW = _NC * _NS
_BPW = BATCH // _NW
_CHUNK = 128
_NCHUNK = _BPW // _CHUNK


def _gather_body(idx_hbm, emb_hbm, out_hbm, idx_v, rows_a, rows_b, sem_a, sem_b):
  wid = lax.axis_index("s") * _NC + lax.axis_index("c")
  base = wid * _BPW
  pltpu.sync_copy(idx_hbm.at[wid], idx_v)
  for c in range(_NCHUNK):
    for j in range(_CHUNK // 16):
      sl = (c, pl.ds(j * 16, 16))
      idx_v[sl] = jnp.clip(idx_v[sl], 0, TOTAL_VOCAB - 1)
  rows = (rows_a, rows_b)
  sems = (sem_a, sem_b)
  copies = []
  for c in range(_NCHUNK):
    copies.append(
        pltpu.async_copy(emb_hbm.at[idx_v.at[c]], rows[c % 2], sems[c % 2]))
    if c >= 1:
      copies[c - 1].wait()
      pltpu.sync_copy(rows[(c - 1) % 2],
                      out_hbm.at[pl.ds(base + (c - 1) * _CHUNK, _CHUNK)])
  copies[_NCHUNK - 1].wait()
  pltpu.sync_copy(rows[(_NCHUNK - 1) % 2],
                  out_hbm.at[pl.ds(base + (_NCHUNK - 1) * _CHUNK, _CHUNK)])


@functools.lru_cache(maxsize=None)
def _build_gather():
  return pl.kernel(
      _gather_body,
      out_type=jax.ShapeDtypeStruct((BATCH, IDENTITY_DIM), jnp.float32),
      mesh=plsc.VectorSubcoreMesh(
          core_axis_name="c", subcore_axis_name="s",
          num_cores=_NC, num_subcores=_NS),
      scratch_types=[
          pltpu.VMEM((_NCHUNK, _CHUNK), jnp.int32),
          pltpu.VMEM((_CHUNK, IDENTITY_DIM), jnp.float32),
          pltpu.VMEM((_CHUNK, IDENTITY_DIM), jnp.float32),
          pltpu.SemaphoreType.DMA,
          pltpu.SemaphoreType.DMA,
      ],
  )

_BM = 2048


def _mlp_body(ident_ref, feat_ref, wf_ref, bf_ref, w1_ref, b1_ref,
              gamma_ref, beta_ref, w2_ref, b2_ref, out_ref):
  fp = jnp.dot(feat_ref[...], wf_ref[...],
               preferred_element_type=jnp.float32) + bf_ref[...][None, :]
  h = (jnp.dot(ident_ref[...], w1_ref[:IDENTITY_DIM, :],
               preferred_element_type=jnp.float32)
       + jnp.dot(fp, w1_ref[IDENTITY_DIM:, :],
                 preferred_element_type=jnp.float32)
       + b1_ref[...][None, :])
  mean = jnp.mean(h, axis=-1, keepdims=True)
  var = jnp.mean(jnp.square(h - mean), axis=-1, keepdims=True)
  h = (h - mean) * lax.rsqrt(var + 1e-5)
  h = h * gamma_ref[...][None, :] + beta_ref[...][None, :]
  h = 0.5 * h * (1.0 + lax.erf(h * (2.0 ** -0.5)))
  out_ref[...] = jnp.dot(h, w2_ref[...],
                         preferred_element_type=jnp.float32) + b2_ref[...][None, :]


def _full(shape):
  return pl.BlockSpec(shape, lambda i: (0,) * len(shape))


_mlp = pl.pallas_call(
    _mlp_body,
    grid=(BATCH // _BM,),
    in_specs=[
        pl.BlockSpec((_BM, IDENTITY_DIM), lambda i: (i, 0)),
        pl.BlockSpec((_BM, FEATURE_DIM), lambda i: (i, 0)),
        _full((FEATURE_DIM, FEATURE_PROJ_DIM)),
        _full((FEATURE_PROJ_DIM,)),
        _full((IDENTITY_DIM + FEATURE_PROJ_DIM, FUSED_DIM)),
        _full((FUSED_DIM,)),
        _full((FUSED_DIM,)),
        _full((FUSED_DIM,)),
        _full((FUSED_DIM, FUSED_DIM)),
        _full((FUSED_DIM,)),
    ],
    out_specs=pl.BlockSpec((_BM, FUSED_DIM), lambda i: (i, 0)),
    out_shape=jax.ShapeDtypeStruct((BATCH, FUSED_DIM), jnp.float32),
    compiler_params=pltpu.CompilerParams(
        dimension_semantics=("parallel",)),
)


@jax.jit
def kernel(drug_id, features, emb, W_feat, b_feat, W1, b1, gamma, beta, W2, b2):
  idx3 = drug_id.reshape(_NW, _NCHUNK, _CHUNK)
  identity = _build_gather()(idx3, emb)
  return _mlp(identity, features,
              W_feat, b_feat, W1, b1, gamma, beta, W2, b2)
